# Initial kernel scaffold; baseline (speedup 1.0000x reference)
import functools

import jax
import jax.numpy as jnp
from jax import lax
from jax.experimental import pallas as pl
from jax.experimental.pallas import tpu as pltpu
from jax.experimental.pallas import tpu_sc as plsc

_N = 10000
_E = 320000
_D = 128
_G = 64
_NC = 2
_NS = 16
_NW = _NC * _NS
_K = 128
_BT = 79
_EPAD = _NW * _BT * _K
_RPT = 626
_NP = _NS * _RPT

_mesh = plsc.VectorSubcoreMesh(core_axis_name="c", subcore_axis_name="s")


@functools.partial(
    pl.kernel,
    out_type=jax.ShapeDtypeStruct((_NC * _NP, 16), jnp.float32),
    mesh=_mesh,
    scratch_types=[
        pltpu.VMEM((_BT, _K), jnp.int32),
        pltpu.VMEM((_K, 16), jnp.float32),
        pltpu.VMEM_SHARED((_NP, 16), jnp.float32),
        pltpu.SemaphoreType.DMA,
    ],
)
def _sc_hist(dst_hbm, ones_hbm, zeros_hbm, out_hbm, didx, ones_v, acc, sem):
    c = lax.axis_index("c")
    s = lax.axis_index("s")
    w = c * _NS + s
    row0 = s * _RPT
    pltpu.sync_copy(dst_hbm.at[w], didx)
    pltpu.sync_copy(ones_hbm, ones_v)
    pltpu.sync_copy(zeros_hbm, acc.at[pl.ds(row0, _RPT)])
    plsc.subcore_barrier()

    def fire(j, carry):
        pltpu.async_copy(ones_v, acc.at[didx.at[j]], sem, add=True)
        return carry

    lax.fori_loop(0, _BT, fire, 0)

    def drain(j, carry):
        pltpu.make_async_copy(ones_v, acc.at[didx.at[j]], sem).wait()
        return carry

    lax.fori_loop(0, _BT, drain, 0)
    plsc.subcore_barrier()
    pltpu.sync_copy(acc.at[pl.ds(row0, _RPT)],
                    out_hbm.at[pl.ds(c * _NP + row0, _RPT)])


@functools.partial(
    pl.kernel,
    out_type=jax.ShapeDtypeStruct((_NC * _NP, _D), jnp.float32),
    mesh=_mesh,
    scratch_types=[
        pltpu.VMEM((_BT, _K), jnp.int32),
        pltpu.VMEM((_BT, _K), jnp.int32),
        pltpu.VMEM((_K, _D), jnp.float32),
        pltpu.VMEM((_K, _D), jnp.float32),
        pltpu.VMEM_SHARED((_NP, _D), jnp.float32),
        pltpu.SemaphoreType.DMA,
    ],
)
def _sc_agg(src_hbm, dst_hbm, y_hbm, zeros_hbm, out_hbm,
            sidx, didx, buf0, buf1, acc, sem):
    c = lax.axis_index("c")
    s = lax.axis_index("s")
    w = c * _NS + s
    row0 = s * _RPT
    pltpu.sync_copy(src_hbm.at[w], sidx)
    pltpu.sync_copy(dst_hbm.at[w], didx)
    pltpu.sync_copy(zeros_hbm, acc.at[pl.ds(row0, _RPT)])
    plsc.subcore_barrier()

    pltpu.async_copy(y_hbm.at[sidx.at[0]], buf0, sem)

    def step(j, buf, nbuf):
        pltpu.make_async_copy(y_hbm.at[sidx.at[j]], buf, sem).wait()

        @pl.when(j + 1 < _BT)
        def _():
            pltpu.async_copy(y_hbm.at[sidx.at[j + 1]], nbuf, sem)

        pltpu.sync_copy(buf, acc.at[didx.at[j]], add=True)

    def body(j, carry):
        @pl.when(j % 2 == 0)
        def _():
            step(j, buf0, buf1)

        @pl.when(j % 2 == 1)
        def _():
            step(j, buf1, buf0)

        return carry

    lax.fori_loop(0, _BT, body, 0)
    plsc.subcore_barrier()
    pltpu.sync_copy(acc.at[pl.ds(row0, _RPT)],
                    out_hbm.at[pl.ds(c * _NP + row0, _RPT)])


def _dis_from_hist(hist_ref):
    deg = hist_ref[0, :_N, 0:1] + hist_ref[1, :_N, 0:1] + 1.0
    return lax.rsqrt(deg)


def _tc1_body(x_ref, w1_ref, hist_ref, y1_ref):
    dis = _dis_from_hist(hist_ref)
    xw = jnp.dot(x_ref[...], w1_ref[...], preferred_element_type=jnp.float32)
    y1_ref[...] = xw * dis


def _tc2_body(hist_ref, p_ref, y_ref, b_ref, w_ref, out_ref):
    dis = _dis_from_hist(hist_ref)
    h = jnp.maximum(
        dis * (p_ref[0, :_N, :] + p_ref[1, :_N, :] + y_ref[...]) + b_ref[...],
        0.0)
    out_ref[...] = jnp.dot(
        h, w_ref[...], preferred_element_type=jnp.float32) * dis


def _tc3_body(hist_ref, p_ref, y_ref, b_ref, batch_ref, wfc_ref, bfc_ref,
              out_ref):
    dis = _dis_from_hist(hist_ref)
    h = jnp.maximum(
        dis * (p_ref[0, :_N, :] + p_ref[1, :_N, :] + y_ref[...]) + b_ref[...],
        0.0)
    oh = (batch_ref[...] == lax.broadcasted_iota(
        jnp.float32, (_N, _G), 1)).astype(jnp.float32)
    sums = lax.dot_general(oh, h, (((0,), (0,)), ((), ())),
                           preferred_element_type=jnp.float32)
    counts = jnp.sum(oh, axis=0)[:, None]
    pooled = sums / jnp.maximum(counts, 1.0)
    out_ref[...] = jnp.dot(
        pooled, wfc_ref[...], preferred_element_type=jnp.float32) + bfc_ref[...]


_tc1 = pl.pallas_call(
    _tc1_body, out_shape=jax.ShapeDtypeStruct((_N, _D), jnp.float32))
_tc2 = pl.pallas_call(
    _tc2_body, out_shape=jax.ShapeDtypeStruct((_N, _D), jnp.float32))
_tc3 = pl.pallas_call(
    _tc3_body, out_shape=jax.ShapeDtypeStruct((_G, _D), jnp.float32))


def kernel(x, edge_index, batch, W1, b1, W2, b2, Wfc, bfc):
    src = edge_index[0]
    dst = edge_index[1]
    npad = _EPAD - _E
    sp = jnp.concatenate(
        [src, jnp.zeros((npad,), jnp.int32)]).reshape(_NW, _BT, _K)
    dp = jnp.concatenate(
        [dst, jnp.full((npad,), _N, jnp.int32)]).reshape(_NW, _BT, _K)
    ones16 = jnp.ones((_K, 16), jnp.float32)
    zeros16 = jnp.zeros((_RPT, 16), jnp.float32)
    zrows = jnp.zeros((_RPT, _D), jnp.float32)

    hist = _sc_hist(dp, ones16, zeros16).reshape(_NC, _NP, 16)
    y1 = _tc1(x, W1, hist)
    p1 = _sc_agg(sp, dp, y1, zrows).reshape(_NC, _NP, _D)
    y2 = _tc2(hist, p1, y1, b1.reshape(1, _D), W2)
    p2 = _sc_agg(sp, dp, y2, zrows).reshape(_NC, _NP, _D)

    out_w = Wfc.shape[1]
    wfc_p = jnp.zeros((_D, _D), jnp.float32).at[:, :out_w].set(Wfc)
    bfc_p = jnp.zeros((1, _D), jnp.float32).at[0, :out_w].set(bfc)
    batch_f = batch.astype(jnp.float32).reshape(_N, 1)
    out = _tc3(hist, p2, y2, b2.reshape(1, _D), batch_f, wfc_p, bfc_p)
    return out[:, :out_w]


# speedup vs baseline: 8.7098x; 8.7098x over previous
import functools

import jax
import jax.numpy as jnp
from jax import lax
from jax.experimental import pallas as pl
from jax.experimental.pallas import tpu as pltpu
from jax.experimental.pallas import tpu_sc as plsc

_N = 10000
_E = 320000
_D = 128
_G = 64
_NC = 2
_NS = 16
_NW = _NC * _NS
_K = 128
_BT = 80
_C = 16
_NCH = _BT // _C
_EPAD = _NW * _BT * _K
_RPT = 632
_NP = _NS * _RPT


@functools.lru_cache(maxsize=1)
def _sc_kernels():
    mesh = plsc.VectorSubcoreMesh(core_axis_name="c", subcore_axis_name="s",
                                  num_cores=_NC, num_subcores=_NS)

    @functools.partial(
        pl.kernel,
        out_type=jax.ShapeDtypeStruct((_NC * _NP, 16), jnp.float32),
        mesh=mesh,
        scratch_types=[
            pltpu.VMEM((_BT, _K), jnp.int32),
            pltpu.VMEM((_K, 16), jnp.float32),
            pltpu.VMEM_SHARED((_NP, 16), jnp.float32),
            pltpu.SemaphoreType.DMA,
        ],
    )
    def sc_hist(dst_hbm, ones_hbm, zeros_hbm, out_hbm, didx, ones_v, acc, sem):
        c = lax.axis_index("c")
        s = lax.axis_index("s")
        w = c * _NS + s
        row0 = s * _RPT
        pltpu.sync_copy(dst_hbm.at[w], didx)
        pltpu.sync_copy(ones_hbm, ones_v)
        pltpu.sync_copy(zeros_hbm, acc.at[pl.ds(row0, _RPT)])
        plsc.subcore_barrier()

        def fire(j, carry):
            pltpu.async_copy(ones_v, acc.at[didx.at[j]], sem, add=True)
            return carry

        lax.fori_loop(0, _BT, fire, 0)

        def drain(j, carry):
            pltpu.make_async_copy(ones_v, acc.at[didx.at[j]], sem).wait()
            return carry

        lax.fori_loop(0, _BT, drain, 0)
        plsc.subcore_barrier()
        pltpu.sync_copy(acc.at[pl.ds(row0, _RPT)],
                        out_hbm.at[pl.ds(c * _NP + row0, _RPT)])

    @functools.partial(
        pl.kernel,
        out_type=jax.ShapeDtypeStruct((_NC * _NP, _D), jnp.float32),
        mesh=mesh,
        scratch_types=[
            pltpu.VMEM((_C, _K), jnp.int32),
            pltpu.VMEM((_C, _K), jnp.int32),
            pltpu.VMEM((_K, _D), jnp.float32),
            pltpu.VMEM((_K, _D), jnp.float32),
            pltpu.VMEM_SHARED((_NP, _D), jnp.float32),
            pltpu.SemaphoreType.DMA,
        ],
    )
    def sc_agg(src_hbm, dst_hbm, y_hbm, zeros_hbm, out_hbm,
               sidx, didx, buf0, buf1, acc, sem):
        c = lax.axis_index("c")
        s = lax.axis_index("s")
        w = c * _NS + s
        row0 = s * _RPT
        pltpu.sync_copy(zeros_hbm, acc.at[pl.ds(row0, _RPT)])
        plsc.subcore_barrier()

        for ch in range(_NCH):
            pltpu.sync_copy(src_hbm.at[w, pl.ds(ch * _C, _C)], sidx)
            pltpu.sync_copy(dst_hbm.at[w, pl.ds(ch * _C, _C)], didx)
            pltpu.async_copy(y_hbm.at[sidx.at[0]], buf0, sem)

            def step(j, buf, nbuf):
                pltpu.make_async_copy(y_hbm.at[sidx.at[j]], buf, sem).wait()

                @pl.when(j + 1 < _C)
                def _():
                    pltpu.async_copy(y_hbm.at[sidx.at[j + 1]], nbuf, sem)

                pltpu.sync_copy(buf, acc.at[didx.at[j]], add=True)

            def body(j, carry):
                @pl.when(j % 2 == 0)
                def _():
                    step(j, buf0, buf1)

                @pl.when(j % 2 == 1)
                def _():
                    step(j, buf1, buf0)

                return carry

            lax.fori_loop(0, _C, body, 0)
        plsc.subcore_barrier()
        pltpu.sync_copy(acc.at[pl.ds(row0, _RPT)],
                        out_hbm.at[pl.ds(c * _NP + row0, _RPT)])

    return sc_hist, sc_agg


def _dis_from_hist(hist_ref):
    deg = hist_ref[0, :_N, 0:1] + hist_ref[1, :_N, 0:1] + 1.0
    return lax.rsqrt(deg)


def _tc1_body(x_ref, w1_ref, hist_ref, y1_ref):
    dis = _dis_from_hist(hist_ref)
    xw = jnp.dot(x_ref[...], w1_ref[...], preferred_element_type=jnp.float32)
    y1_ref[...] = xw * dis


def _tc2_body(hist_ref, p_ref, y_ref, b_ref, w_ref, out_ref):
    dis = _dis_from_hist(hist_ref)
    h = jnp.maximum(
        dis * (p_ref[0, :_N, :] + p_ref[1, :_N, :] + y_ref[...]) + b_ref[...],
        0.0)
    out_ref[...] = jnp.dot(
        h, w_ref[...], preferred_element_type=jnp.float32) * dis


def _tc3_body(hist_ref, p_ref, y_ref, b_ref, batch_ref, wfc_ref, bfc_ref,
              out_ref):
    dis = _dis_from_hist(hist_ref)
    h = jnp.maximum(
        dis * (p_ref[0, :_N, :] + p_ref[1, :_N, :] + y_ref[...]) + b_ref[...],
        0.0)
    gids = lax.broadcasted_iota(jnp.int32, (_N, _G), 1).astype(jnp.float32)
    oh = (batch_ref[...] == gids).astype(jnp.float32)
    sums = lax.dot_general(oh, h, (((0,), (0,)), ((), ())),
                           preferred_element_type=jnp.float32)
    counts = jnp.sum(oh, axis=0)[:, None]
    pooled = sums / jnp.maximum(counts, 1.0)
    out_ref[...] = jnp.dot(
        pooled, wfc_ref[...], preferred_element_type=jnp.float32) + bfc_ref[...]


_tc1 = pl.pallas_call(
    _tc1_body, out_shape=jax.ShapeDtypeStruct((_N, _D), jnp.float32))
_tc2 = pl.pallas_call(
    _tc2_body, out_shape=jax.ShapeDtypeStruct((_N, _D), jnp.float32))
_tc3 = pl.pallas_call(
    _tc3_body, out_shape=jax.ShapeDtypeStruct((_G, _D), jnp.float32))


def kernel(x, edge_index, batch, W1, b1, W2, b2, Wfc, bfc):
    sc_hist, sc_agg = _sc_kernels()
    src = edge_index[0]
    dst = edge_index[1]
    npad = _EPAD - _E
    sp = jnp.concatenate(
        [src, jnp.zeros((npad,), jnp.int32)]).reshape(_NW, _BT, _K)
    dp = jnp.concatenate(
        [dst, jnp.full((npad,), _N, jnp.int32)]).reshape(_NW, _BT, _K)
    ones16 = jnp.ones((_K, 16), jnp.float32)
    zeros16 = jnp.zeros((_RPT, 16), jnp.float32)
    zrows = jnp.zeros((_RPT, _D), jnp.float32)

    hist = sc_hist(dp, ones16, zeros16).reshape(_NC, _NP, 16)
    y1 = _tc1(x, W1, hist)
    p1 = sc_agg(sp, dp, y1, zrows).reshape(_NC, _NP, _D)
    y2 = _tc2(hist, p1, y1, b1.reshape(1, _D), W2)
    p2 = sc_agg(sp, dp, y2, zrows).reshape(_NC, _NP, _D)

    out_w = Wfc.shape[1]
    wfc_p = jnp.zeros((_D, _D), jnp.float32).at[:, :out_w].set(Wfc)
    bfc_p = jnp.zeros((1, _D), jnp.float32).at[0, :out_w].set(bfc)
    batch_f = batch.astype(jnp.float32).reshape(_N, 1)
    out = _tc3(hist, p2, y2, b2.reshape(1, _D), batch_f, wfc_p, bfc_p)
    return out[:, :out_w]
